# baseline (speedup 1.0000x reference)
import functools

import jax
import jax.numpy as jnp
from jax import lax
from jax.experimental import pallas as pl
from jax.experimental.pallas import tpu as pltpu
from jax.experimental.pallas import tpu_sc as plsc

D_MODEL = 1024
SCALE = 32.0
LANES = 16

NUM_CORES = 2
NUM_SUBCORES = 16
NUM_WORKERS = NUM_CORES * NUM_SUBCORES

B_TOTAL = 4 * 4096
B_PER_W = B_TOTAL // NUM_WORKERS
CHUNK = 32
N_CHUNKS = B_PER_W // CHUNK
NBUF = 3
LEAD = 2
HALF = CHUNK // 2

_mesh = plsc.VectorSubcoreMesh(core_axis_name="c", subcore_axis_name="s")


@functools.partial(
    pl.kernel,
    out_type=jax.ShapeDtypeStruct((B_TOTAL, D_MODEL), jnp.float32),
    mesh=_mesh,
    scratch_types=[
        pltpu.VMEM((B_PER_W,), jnp.int32),
        [pltpu.VMEM((CHUNK, D_MODEL), jnp.float32) for _ in range(NBUF)],
        [pltpu.SemaphoreType.DMA for _ in range(NBUF)],
        [pltpu.SemaphoreType.DMA for _ in range(NBUF)],
    ],
)
def _embed_lookup(lut_hbm, idx_hbm, out_hbm, idx_v, bufs, gsems, ssems):
    wid = lax.axis_index("s") * NUM_CORES + lax.axis_index("c")
    base = wid * B_PER_W
    pltpu.sync_copy(idx_hbm.at[pl.ds(base, B_PER_W)], idx_v)

    def gather_start(c, b):
        pltpu.async_copy(
            lut_hbm.at[idx_v.at[pl.ds(c * CHUNK, CHUNK)]], bufs[b], gsems[b]
        )

    def out_slice(c):
        return out_hbm.at[pl.ds(pl.multiple_of(base + c * CHUNK, 8), CHUNK)]

    def out_half(c, h):
        return out_hbm.at[
            pl.ds(pl.multiple_of(base + c * CHUNK + h * HALF, 8), HALF)
        ]

    for b in range(LEAD):
        gather_start(b, b)

    for c in range(N_CHUNKS):
        b = c % NBUF
        pltpu.make_async_copy(
            lut_hbm.at[idx_v.at[pl.ds(c * CHUNK, CHUNK)]], bufs[b], gsems[b]
        ).wait()

        cp = c + LEAD
        bp = cp % NBUF
        if cp < N_CHUNKS:
            if cp >= NBUF:
                pltpu.make_async_copy(
                    bufs[bp], out_slice(cp - NBUF), ssems[bp]
                ).wait()
            gather_start(cp, bp)

        for h in range(2):

            def row_body(r, carry2, _b=b):
                for j in range(D_MODEL // LANES):
                    bufs[_b][r, pl.ds(j * LANES, LANES)] = (
                        bufs[_b][r, pl.ds(j * LANES, LANES)] * SCALE
                    )
                return carry2

            lax.fori_loop(h * HALF, (h + 1) * HALF, row_body, 0)
            pltpu.async_copy(
                bufs[b].at[pl.ds(h * HALF, HALF)], out_half(c, h), ssems[b]
            )

    for c in range(N_CHUNKS - NBUF, N_CHUNKS):
        pltpu.make_async_copy(bufs[c % NBUF], out_slice(c), ssems[c % NBUF]).wait()


def kernel(x, lut):
    idx = x.reshape(-1).astype(jnp.int32)
    out = _embed_lookup(lut, idx)
    return out.reshape(x.shape + (D_MODEL,))


# speedup vs baseline: 1.1387x; 1.1387x over previous
import functools

import jax
import jax.numpy as jnp
from jax import lax
from jax.experimental import pallas as pl
from jax.experimental.pallas import tpu as pltpu
from jax.experimental.pallas import tpu_sc as plsc

D_MODEL = 1024
SCALE = 32.0
LANES = 16

NUM_CORES = 2
NUM_SUBCORES = 16
NUM_WORKERS = NUM_CORES * NUM_SUBCORES

B_TOTAL = 4 * 4096
B_PER_W = B_TOTAL // NUM_WORKERS
CHUNK = 8
N_CHUNKS = B_PER_W // CHUNK
NBUF = 8
LEAD = 4
N_GROUPS = N_CHUNKS // NBUF
SPLIT_HALVES = CHUNK >= 16

_mesh = plsc.VectorSubcoreMesh(core_axis_name="c", subcore_axis_name="s")


@functools.partial(
    pl.kernel,
    out_type=jax.ShapeDtypeStruct((B_TOTAL, D_MODEL), jnp.float32),
    mesh=_mesh,
    scratch_types=[
        pltpu.VMEM((B_PER_W,), jnp.int32),
        [pltpu.VMEM((CHUNK, D_MODEL), jnp.float32) for _ in range(NBUF)],
        [pltpu.SemaphoreType.DMA for _ in range(NBUF)],
        [pltpu.SemaphoreType.DMA for _ in range(NBUF)],
    ],
)
def _embed_lookup(lut_hbm, idx_hbm, out_hbm, idx_v, bufs, gsems, ssems):
    wid = lax.axis_index("s") * NUM_CORES + lax.axis_index("c")
    base = wid * B_PER_W
    pltpu.sync_copy(idx_hbm.at[pl.ds(base, B_PER_W)], idx_v)

    def gather_start(c, b):
        pltpu.async_copy(
            lut_hbm.at[idx_v.at[pl.ds(c * CHUNK, CHUNK)]], bufs[b], gsems[b]
        )

    def out_rows(c, lo, n):
        return out_hbm.at[pl.ds(pl.multiple_of(base + c * CHUNK + lo, 8), n)]

    for b in range(LEAD):
        gather_start(b, b)

    def scale_rows(b, lo, hi):
        def row_body(r, carry, _b=b):
            for j in range(D_MODEL // LANES):
                bufs[_b][r, pl.ds(j * LANES, LANES)] = (
                    bufs[_b][r, pl.ds(j * LANES, LANES)] * SCALE
                )
            return carry

        lax.fori_loop(lo, hi, row_body, 0)

    def group_body(g, carry):
        for b in range(NBUF):
            c = g * NBUF + b
            pltpu.make_async_copy(
                lut_hbm.at[idx_v.at[pl.ds(c * CHUNK, CHUNK)]], bufs[b], gsems[b]
            ).wait()

            cp = c + LEAD
            bp = (b + LEAD) % NBUF

            @pl.when(jnp.logical_and(cp >= NBUF, cp < N_CHUNKS))
            def _wait_prev_scatter(_cp=cp, _bp=bp):
                pltpu.make_async_copy(
                    bufs[_bp], out_rows(_cp - NBUF, 0, CHUNK), ssems[_bp]
                ).wait()

            @pl.when(cp < N_CHUNKS)
            def _prefetch(_cp=cp, _bp=bp):
                gather_start(_cp, _bp)

            if SPLIT_HALVES:
                half = CHUNK // 2
                for h in range(2):
                    scale_rows(b, h * half, (h + 1) * half)
                    pltpu.async_copy(
                        bufs[b].at[pl.ds(h * half, half)],
                        out_rows(c, h * half, half),
                        ssems[b],
                    )
            else:
                scale_rows(b, 0, CHUNK)
                pltpu.async_copy(bufs[b], out_rows(c, 0, CHUNK), ssems[b])

        return carry

    lax.fori_loop(0, N_GROUPS, group_body, 0)

    for b in range(NBUF):
        c = N_CHUNKS - NBUF + b
        pltpu.make_async_copy(bufs[b], out_rows(c, 0, CHUNK), ssems[b]).wait()


def kernel(x, lut):
    idx = x.reshape(-1).astype(jnp.int32)
    out = _embed_lookup(lut, idx)
    return out.reshape(x.shape + (D_MODEL,))


# speedup vs baseline: 1.1547x; 1.0140x over previous
import functools

import jax
import jax.numpy as jnp
from jax import lax
from jax.experimental import pallas as pl
from jax.experimental.pallas import tpu as pltpu
from jax.experimental.pallas import tpu_sc as plsc

D_MODEL = 1024
SCALE = 32.0
LANES = 16

NUM_CORES = 2
NUM_SUBCORES = 16
NUM_WORKERS = NUM_CORES * NUM_SUBCORES

B_TOTAL = 4 * 4096
B_PER_W = B_TOTAL // NUM_WORKERS
CHUNK = 8
N_CHUNKS = B_PER_W // CHUNK
NBUF = 8
LEAD = 6
N_GROUPS = N_CHUNKS // NBUF
SPLIT_HALVES = CHUNK >= 16

_mesh = plsc.VectorSubcoreMesh(core_axis_name="c", subcore_axis_name="s")


@functools.partial(
    pl.kernel,
    out_type=jax.ShapeDtypeStruct((B_TOTAL, D_MODEL), jnp.float32),
    mesh=_mesh,
    scratch_types=[
        pltpu.VMEM((B_PER_W,), jnp.int32),
        [pltpu.VMEM((CHUNK, D_MODEL), jnp.float32) for _ in range(NBUF)],
        [pltpu.SemaphoreType.DMA for _ in range(NBUF)],
        [pltpu.SemaphoreType.DMA for _ in range(NBUF)],
    ],
)
def _embed_lookup(lut_hbm, idx_hbm, out_hbm, idx_v, bufs, gsems, ssems):
    wid = lax.axis_index("s") * NUM_CORES + lax.axis_index("c")
    base = wid * B_PER_W
    pltpu.sync_copy(idx_hbm.at[pl.ds(base, B_PER_W)], idx_v)

    def gather_start(c, b):
        pltpu.async_copy(
            lut_hbm.at[idx_v.at[pl.ds(c * CHUNK, CHUNK)]], bufs[b], gsems[b]
        )

    def out_rows(c, lo, n):
        return out_hbm.at[pl.ds(pl.multiple_of(base + c * CHUNK + lo, 8), n)]

    for b in range(LEAD):
        gather_start(b, b)

    def scale_rows(b, lo, hi):
        def row_body(r, carry, _b=b):
            for j in range(D_MODEL // LANES):
                bufs[_b][r, pl.ds(j * LANES, LANES)] = (
                    bufs[_b][r, pl.ds(j * LANES, LANES)] * SCALE
                )
            return carry

        lax.fori_loop(lo, hi, row_body, 0)

    def group_body(g, carry):
        for b in range(NBUF):
            c = g * NBUF + b
            pltpu.make_async_copy(
                lut_hbm.at[idx_v.at[pl.ds(c * CHUNK, CHUNK)]], bufs[b], gsems[b]
            ).wait()

            cp = c + LEAD
            bp = (b + LEAD) % NBUF

            @pl.when(jnp.logical_and(cp >= NBUF, cp < N_CHUNKS))
            def _wait_prev_scatter(_cp=cp, _bp=bp):
                pltpu.make_async_copy(
                    bufs[_bp], out_rows(_cp - NBUF, 0, CHUNK), ssems[_bp]
                ).wait()

            @pl.when(cp < N_CHUNKS)
            def _prefetch(_cp=cp, _bp=bp):
                gather_start(_cp, _bp)

            if SPLIT_HALVES:
                half = CHUNK // 2
                for h in range(2):
                    scale_rows(b, h * half, (h + 1) * half)
                    pltpu.async_copy(
                        bufs[b].at[pl.ds(h * half, half)],
                        out_rows(c, h * half, half),
                        ssems[b],
                    )
            else:
                scale_rows(b, 0, CHUNK)
                pltpu.async_copy(bufs[b], out_rows(c, 0, CHUNK), ssems[b])

        return carry

    lax.fori_loop(0, N_GROUPS, group_body, 0)

    for b in range(NBUF):
        c = N_CHUNKS - NBUF + b
        pltpu.make_async_copy(bufs[b], out_rows(c, 0, CHUNK), ssems[b]).wait()


def kernel(x, lut):
    idx = x.reshape(-1).astype(jnp.int32)
    out = _embed_lookup(lut, idx)
    return out.reshape(x.shape + (D_MODEL,))
